# baseline (speedup 1.0000x reference)
import jax
import jax.numpy as jnp
from jax.experimental import pallas as pl
from jax.experimental.pallas import tpu as pltpu

_B, _S, _C = 4096, 50, 256
_RB = 64
_G = _B // _RB
_NBUF = 4


def _onehot_block(x_ref, out_ref, *scratch):
    sbufs = scratch[:_NBUF]
    sems = scratch[_NBUF:]
    i = pl.program_id(0)

    def copy(j, s):
        return pltpu.make_async_copy(
            sbufs[s], out_ref.at[pl.ds(j * _RB, _RB), :, :], sems[s]
        )

    idx = x_ref[...]
    iota = jax.lax.broadcasted_iota(jnp.int32, (_RB, _S, _C), 2)
    block = (idx[:, :, None] == iota).astype(jnp.float32)

    for s in range(_NBUF):
        @pl.when(jax.lax.rem(i, _NBUF) == s)
        def _go(s=s):
            @pl.when(i >= _NBUF)
            def _wait_prev():
                copy(i - _NBUF, s).wait()

            sbufs[s][...] = block
            copy(i, s).start()

    @pl.when(i == _G - 1)
    def _drain():
        for d in range(_NBUF):
            j = _G - _NBUF + d
            copy(j, j % _NBUF).wait()


def kernel(x):
    return pl.pallas_call(
        _onehot_block,
        grid=(_G,),
        in_specs=[pl.BlockSpec((_RB, _S), lambda i: (i, 0))],
        out_specs=pl.BlockSpec(memory_space=pltpu.MemorySpace.HBM),
        out_shape=jax.ShapeDtypeStruct((_B, _S, _C), jnp.float32),
        scratch_shapes=(
            [pltpu.VMEM((_RB, _S, _C), jnp.float32) for _ in range(_NBUF)]
            + [pltpu.SemaphoreType.DMA for _ in range(_NBUF)]
        ),
        compiler_params=pltpu.CompilerParams(
            dimension_semantics=("arbitrary",),
        ),
    )(x.astype(jnp.int32))
